# baseline (speedup 1.0000x reference)
import jax
import jax.numpy as jnp
from jax import lax
from jax.experimental import pallas as pl
from jax.experimental.pallas import tpu as pltpu
from jax.experimental.pallas import tpu_sc as plsc

_NC = 2
_NS = 16
_NW = _NC * _NS
_L = 16
_D = 32
_B = 16384
_BPW = _B // _NW
_CH = 128
_NCH = _BPW // _CH
_GPC = _CH // _L


def _body(p1_hbm, p2_hbm, P_hbm, Q_hbm, b1_hbm, b2_hbm, out_hbm,
          idx1_v, idx2_v, p_v, q_v, b1_v, b2_v, out_v,
          sem0, sem1, sem2, sem3):
    wid = lax.axis_index("s") * _NC + lax.axis_index("c")
    sems = (sem0, sem1, sem2, sem3)

    pltpu.sync_copy(p1_hbm.at[pl.ds(wid * _NCH, _NCH)], idx1_v)
    pltpu.sync_copy(p2_hbm.at[pl.ds(wid * _NCH, _NCH)], idx2_v)

    copies = []
    for c in range(_NCH):
        sem = sems[c]
        copies.append((
            pltpu.async_copy(P_hbm.at[idx1_v.at[c]], p_v.at[c], sem),
            pltpu.async_copy(Q_hbm.at[idx2_v.at[c]], q_v.at[c], sem),
            pltpu.async_copy(b1_hbm.at[idx1_v.at[c]], b1_v.at[c], sem),
            pltpu.async_copy(b2_hbm.at[idx2_v.at[c]], b2_v.at[c], sem),
        ))

    lane = lax.iota(jnp.int32, _L)

    for c in range(_NCH):
        for cp in copies[c]:
            cp.wait()
        pb, qb = p_v.at[c], q_v.at[c]

        def group(g, carry):
            rows = g * _L + lane
            acc = plsc.load_gather(pb, [rows, jnp.zeros((_L,), jnp.int32)]) * \
                plsc.load_gather(qb, [rows, jnp.zeros((_L,), jnp.int32)])
            for j in range(1, _D):
                cj = jnp.full((_L,), j, dtype=jnp.int32)
                acc = acc + plsc.load_gather(pb, [rows, cj]) * \
                    plsc.load_gather(qb, [rows, cj])
            sl = pl.ds(g * _L, _L)
            acc = acc + b1_v.at[c][sl] + b2_v.at[c][sl]
            out_v[pl.ds(c * _CH + g * _L, _L)] = acc
            return carry

        lax.fori_loop(0, _GPC, group, 0)

    pltpu.sync_copy(out_v, out_hbm.at[pl.ds(wid * _BPW, _BPW)])


@jax.jit
def kernel(player1, player2, P, Q, player1_bias, player2_bias):
    p1 = player1.astype(jnp.int32).reshape(_NW * _NCH, _CH)
    p2 = player2.astype(jnp.int32).reshape(_NW * _NCH, _CH)
    b1 = player1_bias.reshape(-1)
    b2 = player2_bias.reshape(-1)
    mesh = plsc.VectorSubcoreMesh(core_axis_name="c", subcore_axis_name="s")

    f = pl.kernel(
        _body,
        out_type=jax.ShapeDtypeStruct((_B,), jnp.float32),
        mesh=mesh,
        compiler_params=pltpu.CompilerParams(
            needs_layout_passes=False, use_tc_tiling_on_sc=False),
        scratch_types=[
            pltpu.VMEM((_NCH, _CH), jnp.int32),
            pltpu.VMEM((_NCH, _CH), jnp.int32),
            pltpu.VMEM((_NCH, _CH, _D), jnp.float32),
            pltpu.VMEM((_NCH, _CH, _D), jnp.float32),
            pltpu.VMEM((_NCH, _CH), jnp.float32),
            pltpu.VMEM((_NCH, _CH), jnp.float32),
            pltpu.VMEM((_BPW,), jnp.float32),
            pltpu.SemaphoreType.DMA,
            pltpu.SemaphoreType.DMA,
            pltpu.SemaphoreType.DMA,
            pltpu.SemaphoreType.DMA,
        ],
    )

    return f(p1, p2, P, Q, b1, b2)


# speedup vs baseline: 1.3436x; 1.3436x over previous
import jax
import jax.numpy as jnp
from jax import lax
from jax.experimental import pallas as pl
from jax.experimental.pallas import tpu as pltpu
from jax.experimental.pallas import tpu_sc as plsc

_NC = 2
_NS = 16
_NW = _NC * _NS
_L = 16
_D = 32
_B = 16384
_BPW = _B // _NW
_CH = 128
_NCH = _BPW // _CH
_GPC = _CH // _L


def _dot_body(p1_hbm, p2_hbm, P_hbm, Q_hbm, out_hbm,
              idx1_v, idx2_v, p_v, q_v, out_v, sem0, sem1):
    wid = lax.axis_index("s") * _NC + lax.axis_index("c")
    base = wid * _BPW
    sems = (sem0, sem1)

    pltpu.sync_copy(p1_hbm.at[pl.ds(base, _BPW)], idx1_v)
    pltpu.sync_copy(p2_hbm.at[pl.ds(base, _BPW)], idx2_v)

    lane = lax.iota(jnp.int32, _L)

    def fire(c):
        buf = c % 2
        sem = sems[buf]

        def fire_group(g, carry):
            sl = pl.ds(c * _CH + g * _L, _L)
            v1 = idx1_v[sl]
            v2 = idx2_v[sl]
            for i in range(_L):
                e = g * _L + i
                pltpu.async_copy(P_hbm.at[pl.ds(v1[i], 1)],
                                 p_v.at[buf].at[pl.ds(e, 1)], sem)
                pltpu.async_copy(Q_hbm.at[pl.ds(v2[i], 1)],
                                 q_v.at[buf].at[pl.ds(e, 1)], sem)
            return carry

        lax.fori_loop(0, _GPC, fire_group, 0)

    def drain(c):
        buf = c % 2
        sem = sems[buf]
        pltpu.make_async_copy(P_hbm.at[pl.ds(0, _CH)], p_v.at[buf], sem).wait()
        pltpu.make_async_copy(Q_hbm.at[pl.ds(0, _CH)], q_v.at[buf], sem).wait()

    fire(0)
    fire(1)

    for c in range(_NCH):
        buf = c % 2
        drain(c)
        pb, qb = p_v.at[buf], q_v.at[buf]

        def group(g, carry):
            sl = pl.ds(c * _CH + g * _L, _L)
            rows = g * _L + lane
            acc = plsc.load_gather(pb, [rows, jnp.zeros((_L,), jnp.int32)]) * \
                plsc.load_gather(qb, [rows, jnp.zeros((_L,), jnp.int32)])
            for j in range(1, _D):
                cj = jnp.full((_L,), j, dtype=jnp.int32)
                acc = acc + plsc.load_gather(pb, [rows, cj]) * \
                    plsc.load_gather(qb, [rows, cj])
            out_v[sl] = acc
            return carry

        lax.fori_loop(0, _GPC, group, 0)

        if c + 2 < _NCH:
            fire(c + 2)

    pltpu.sync_copy(out_v, out_hbm.at[pl.ds(base, _BPW)])


def _bias_body(p1_hbm, p2_hbm, b1_hbm, b2_hbm, out_hbm,
               idx1_v, idx2_v, b1_v, b2_v, out_v, sem):
    wid = lax.axis_index("s") * _NC + lax.axis_index("c")
    nrow = _BPW // 128

    pltpu.sync_copy(p1_hbm.at[pl.ds(wid * nrow, nrow)], idx1_v)
    pltpu.sync_copy(p2_hbm.at[pl.ds(wid * nrow, nrow)], idx2_v)

    copies = []
    for c in range(nrow):
        sl = pl.ds(c * 128, 128)
        copies.append(pltpu.async_copy(b1_hbm.at[idx1_v.at[c]], b1_v.at[sl], sem))
        copies.append(pltpu.async_copy(b2_hbm.at[idx2_v.at[c]], b2_v.at[sl], sem))
    for cp in copies:
        cp.wait()

    for g in range(_BPW // _L):
        sl = pl.ds(g * _L, _L)
        out_v[sl] = b1_v[sl] + b2_v[sl]

    pltpu.sync_copy(out_v, out_hbm.at[pl.ds(wid * _BPW, _BPW)])


@jax.jit
def kernel(player1, player2, P, Q, player1_bias, player2_bias):
    p1 = player1.astype(jnp.int32)
    p2 = player2.astype(jnp.int32)
    b1 = player1_bias.reshape(-1)
    b2 = player2_bias.reshape(-1)
    mesh = plsc.VectorSubcoreMesh(core_axis_name="c", subcore_axis_name="s")

    dot_f = pl.kernel(
        _dot_body,
        out_type=jax.ShapeDtypeStruct((_B,), jnp.float32),
        mesh=mesh,
        compiler_params=pltpu.CompilerParams(needs_layout_passes=False),
        scratch_types=[
            pltpu.VMEM((_BPW,), jnp.int32),
            pltpu.VMEM((_BPW,), jnp.int32),
            pltpu.VMEM((2, _CH, _D), jnp.float32),
            pltpu.VMEM((2, _CH, _D), jnp.float32),
            pltpu.VMEM((_BPW,), jnp.float32),
            pltpu.SemaphoreType.DMA,
            pltpu.SemaphoreType.DMA,
        ],
    )

    bias_f = pl.kernel(
        _bias_body,
        out_type=jax.ShapeDtypeStruct((_B,), jnp.float32),
        mesh=mesh,
        compiler_params=pltpu.CompilerParams(
            needs_layout_passes=False, use_tc_tiling_on_sc=False),
        scratch_types=[
            pltpu.VMEM((_BPW // 128, 128), jnp.int32),
            pltpu.VMEM((_BPW // 128, 128), jnp.int32),
            pltpu.VMEM((_BPW,), jnp.float32),
            pltpu.VMEM((_BPW,), jnp.float32),
            pltpu.VMEM((_BPW,), jnp.float32),
            pltpu.SemaphoreType.DMA,
        ],
    )

    dots = dot_f(p1, p2, P, Q)
    biases = bias_f(p1.reshape(_NW * (_BPW // 128), 128),
                    p2.reshape(_NW * (_BPW // 128), 128), b1, b2)
    return dots + biases
